# baseline (speedup 1.0000x reference)
import functools

import jax
import jax.numpy as jnp
from jax import lax
from jax.experimental import pallas as pl
from jax.experimental.pallas import tpu as pltpu
from jax.experimental.pallas import tpu_sc as plsc

_N = 100000
_C = 16
_K = 27
_CH = 128
_NC = 2
_NS = 16
_NW = _NC * _NS
_CPW = 25
_NPAD = _NW * _CPW * _CH
_R = _NPAD * _K
_ZROW = _N * _K
_EPS = 1e-3
_BN = 1024


def _mm_body(x_ref, w_ref, o_ref):
    o_ref[...] = jnp.dot(x_ref[...], w_ref[...], preferred_element_type=jnp.float32)


def _mm(x, wflat):
    return pl.pallas_call(
        _mm_body,
        grid=(_NPAD // _BN,),
        in_specs=[
            pl.BlockSpec((_BN, _C), lambda i: (i, 0)),
            pl.BlockSpec((_C, _K * _C), lambda i: (0, 0)),
        ],
        out_specs=pl.BlockSpec((_BN, _K * _C), lambda i: (i, 0)),
        out_shape=jax.ShapeDtypeStruct((_NPAD, _K * _C), jnp.float32),
    )(x, wflat)


def _affine_from_partials(ps, pq, g, b):
    s = jnp.sum(ps, axis=0, keepdims=True)
    q = jnp.sum(pq, axis=0, keepdims=True)
    m = s / _N
    v = q / _N - m * m
    inv = lax.rsqrt(v + _EPS)
    a = g * inv
    c = b - m * a
    return a, c


def _bnmm_body(x_ref, ps_ref, pq_ref, g_ref, b_ref, w_ref, o_ref):
    a, c = _affine_from_partials(ps_ref[...], pq_ref[...], g_ref[...], b_ref[...])
    h = jnp.maximum(x_ref[...] * a + c, 0.0)
    rows = pl.program_id(0) * _BN + lax.broadcasted_iota(jnp.int32, (_BN, 1), 0)
    h = jnp.where(rows < _N, h, 0.0)
    o_ref[...] = jnp.dot(h, w_ref[...], preferred_element_type=jnp.float32)


def _bnmm(x, ps, pq, g, b, wflat):
    return pl.pallas_call(
        _bnmm_body,
        grid=(_NPAD // _BN,),
        in_specs=[
            pl.BlockSpec((_BN, _C), lambda i: (i, 0)),
            pl.BlockSpec((_NW, _C), lambda i: (0, 0)),
            pl.BlockSpec((_NW, _C), lambda i: (0, 0)),
            pl.BlockSpec((1, _C), lambda i: (0, 0)),
            pl.BlockSpec((1, _C), lambda i: (0, 0)),
            pl.BlockSpec((_C, _K * _C), lambda i: (0, 0)),
        ],
        out_specs=pl.BlockSpec((_BN, _K * _C), lambda i: (i, 0)),
        out_shape=jax.ShapeDtypeStruct((_NPAD, _K * _C), jnp.float32),
    )(x, ps, pq, g, b, wflat)


def _final_body(x_ref, ps_ref, pq_ref, g_ref, b_ref, f_ref, o_ref):
    a, c = _affine_from_partials(ps_ref[...], pq_ref[...], g_ref[...], b_ref[...])
    o_ref[...] = jnp.maximum(x_ref[...] * a + c + f_ref[...], 0.0)


def _final(x, ps, pq, g, b, f):
    return pl.pallas_call(
        _final_body,
        grid=(_NPAD // _BN,),
        in_specs=[
            pl.BlockSpec((_BN, _C), lambda i: (i, 0)),
            pl.BlockSpec((_NW, _C), lambda i: (0, 0)),
            pl.BlockSpec((_NW, _C), lambda i: (0, 0)),
            pl.BlockSpec((1, _C), lambda i: (0, 0)),
            pl.BlockSpec((1, _C), lambda i: (0, 0)),
            pl.BlockSpec((_BN, _C), lambda i: (i, 0)),
        ],
        out_specs=pl.BlockSpec((_BN, _C), lambda i: (i, 0)),
        out_shape=jax.ShapeDtypeStruct((_NPAD, _C), jnp.float32),
    )(x, ps, pq, g, b, f)


def _sc_body(g3, idxh, out_h, ps_h, pq_h, idx_v, rows_v, out_v, st_v, sem):
    wid = lax.axis_index("s") * _NC + lax.axis_index("c")

    def chunk_body(c, stats):
        g = wid * _CPW + c
        pltpu.sync_copy(idxh.at[g], idx_v)

        def fire(j, carry):
            pltpu.async_copy(g3.at[idx_v.at[j]], rows_v.at[pl.ds(j * _CH, _CH)], sem)
            return carry

        lax.fori_loop(0, _K, fire, 0)
        pltpu.make_async_copy(g3.at[pl.ds(0, _K * _CH)], rows_v, sem).wait()

        def nbody(ln, carry):
            sv, qv = carry
            base = ln * _K
            acc = rows_v[base, :]
            for k in range(1, _K):
                acc = acc + rows_v[base + k, :]
            out_v[ln, :] = acc
            return (sv + acc, qv + acc * acc)

        sv, qv = lax.fori_loop(0, _CH, nbody, stats)
        pltpu.sync_copy(out_v, out_h.at[pl.ds(g * _CH, _CH)])
        return (sv, qv)

    z = jnp.zeros((_C,), jnp.float32)
    sv, qv = lax.fori_loop(0, _CPW, chunk_body, (z, z))
    st_v[0, :] = sv
    st_v[1, :] = qv
    pltpu.sync_copy(st_v.at[pl.ds(0, 1)], ps_h.at[pl.ds(wid, 1)])
    pltpu.sync_copy(st_v.at[pl.ds(1, 1)], pq_h.at[pl.ds(wid, 1)])


@functools.cache
def _sc_gather_sum_kernel():
    return functools.partial(
        pl.kernel,
        out_type=[
            jax.ShapeDtypeStruct((_NPAD, _C), jnp.float32),
            jax.ShapeDtypeStruct((_NW, _C), jnp.float32),
            jax.ShapeDtypeStruct((_NW, _C), jnp.float32),
        ],
        mesh=plsc.VectorSubcoreMesh(
            core_axis_name="c", subcore_axis_name="s",
            num_cores=_NC, num_subcores=_NS),
        scratch_types=[
            pltpu.VMEM((_K, _CH), jnp.int32),
            pltpu.VMEM((_K * _CH, _C), jnp.float32),
            pltpu.VMEM((_CH, _C), jnp.float32),
            pltpu.VMEM((2, _C), jnp.float32),
            pltpu.SemaphoreType.DMA,
        ],
        compiler_params=pltpu.CompilerParams(use_tc_tiling_on_sc=False),
    )(_sc_body)


def kernel(features, neighbor_idx, W1, W2, gamma1, beta1, gamma2, beta2):
    nbr = neighbor_idx.astype(jnp.int32)
    k_off = jnp.arange(_K, dtype=jnp.int32)[None, :]
    cidx = jnp.where(nbr >= 0, nbr * _K + k_off, _ZROW)
    cidx = jnp.pad(cidx, ((0, _NPAD - _N), (0, 0)), constant_values=_ZROW)
    idx3d = cidx.reshape(-1, _K, _CH)

    f_pad = jnp.pad(features, ((0, _NPAD - _N), (0, 0)))
    w1f = W1.transpose(1, 0, 2).reshape(_C, _K * _C)
    w2f = W2.transpose(1, 0, 2).reshape(_C, _K * _C)
    g1v = gamma1.reshape(1, _C)
    b1v = beta1.reshape(1, _C)
    g2v = gamma2.reshape(1, _C)
    b2v = beta2.reshape(1, _C)

    g1 = _mm(f_pad, w1f).reshape(_R, _C)
    out1, ps1, pq1 = _sc_gather_sum_kernel()(g1, idx3d)
    g2 = _bnmm(out1, ps1, pq1, g1v, b1v, w2f).reshape(_R, _C)
    out2, ps2, pq2 = _sc_gather_sum_kernel()(g2, idx3d)
    y = _final(out2, ps2, pq2, g2v, b2v, f_pad)
    return y[:_N]


# speedup vs baseline: 9.3935x; 9.3935x over previous
import functools

import jax
import jax.numpy as jnp
from jax import lax
from jax.experimental import pallas as pl
from jax.experimental.pallas import tpu as pltpu
from jax.experimental.pallas import tpu_sc as plsc

_N = 100000
_C = 16
_K = 27
_CH = 64
_NC = 2
_NS = 16
_NW = _NC * _NS
_CPW = 50
_NPAD = _NW * _CPW * _CH
_R = _NPAD * _K
_TROWS = _N + 8
_EPS = 1e-3
_BN = 1024
_GRID = _NPAD // _BN


def _mm_stats_body(g_ref, w_ref, o_ref, st_ref, acc_ref):
    i = pl.program_id(0)
    out = jnp.dot(g_ref[...], w_ref[...], preferred_element_type=jnp.float32)
    o_ref[...] = out

    @pl.when(i == 0)
    def _():
        acc_ref[...] = jnp.zeros((2, _C), jnp.float32)

    s = jnp.sum(out, axis=0, keepdims=True)
    q = jnp.sum(out * out, axis=0, keepdims=True)
    acc_ref[...] = acc_ref[...] + jnp.concatenate([s, q], axis=0)

    @pl.when(i == _GRID - 1)
    def _():
        st_ref[...] = acc_ref[...]


def _mm_stats(gth, wstk):
    return pl.pallas_call(
        _mm_stats_body,
        grid=(_GRID,),
        in_specs=[
            pl.BlockSpec((_BN, _K * _C), lambda i: (i, 0)),
            pl.BlockSpec((_K * _C, _C), lambda i: (0, 0)),
        ],
        out_specs=[
            pl.BlockSpec((_BN, _C), lambda i: (i, 0)),
            pl.BlockSpec((2, _C), lambda i: (0, 0)),
        ],
        out_shape=[
            jax.ShapeDtypeStruct((_NPAD, _C), jnp.float32),
            jax.ShapeDtypeStruct((2, _C), jnp.float32),
        ],
        scratch_shapes=[pltpu.VMEM((2, _C), jnp.float32)],
    )(gth, wstk)


def _affine_from_stats(st, g, b):
    m = st[0:1, :] / _N
    v = st[1:2, :] / _N - m * m
    inv = lax.rsqrt(v + _EPS)
    a = g * inv
    c = b - m * a
    return a, c


def _affine_relu_body(x_ref, st_ref, g_ref, b_ref, o_ref):
    a, c = _affine_from_stats(st_ref[...], g_ref[...], b_ref[...])
    h = jnp.maximum(x_ref[...] * a + c, 0.0)
    rows = pl.program_id(0) * _BN + lax.broadcasted_iota(jnp.int32, (_BN, 1), 0)
    o_ref[...] = jnp.where(rows < _N, h, 0.0)


def _affine_relu(x, st, g, b):
    return pl.pallas_call(
        _affine_relu_body,
        grid=(_GRID,),
        in_specs=[
            pl.BlockSpec((_BN, _C), lambda i: (i, 0)),
            pl.BlockSpec((2, _C), lambda i: (0, 0)),
            pl.BlockSpec((1, _C), lambda i: (0, 0)),
            pl.BlockSpec((1, _C), lambda i: (0, 0)),
        ],
        out_specs=pl.BlockSpec((_BN, _C), lambda i: (i, 0)),
        out_shape=jax.ShapeDtypeStruct((_NPAD, _C), jnp.float32),
    )(x, st, g, b)


def _final_body(x_ref, st_ref, g_ref, b_ref, f_ref, o_ref):
    a, c = _affine_from_stats(st_ref[...], g_ref[...], b_ref[...])
    o_ref[...] = jnp.maximum(x_ref[...] * a + c + f_ref[...], 0.0)


def _final(x, st, g, b, f):
    return pl.pallas_call(
        _final_body,
        grid=(_GRID,),
        in_specs=[
            pl.BlockSpec((_BN, _C), lambda i: (i, 0)),
            pl.BlockSpec((2, _C), lambda i: (0, 0)),
            pl.BlockSpec((1, _C), lambda i: (0, 0)),
            pl.BlockSpec((1, _C), lambda i: (0, 0)),
            pl.BlockSpec((_BN, _C), lambda i: (i, 0)),
        ],
        out_specs=pl.BlockSpec((_BN, _C), lambda i: (i, 0)),
        out_shape=jax.ShapeDtypeStruct((_NPAD, _C), jnp.float32),
    )(x, st, g, b, f)


def _sc_body(f_hbm, idxh, gth_hbm, f_sp, idx_v, gth_v, sem):
    sid = lax.axis_index("s")
    cid = lax.axis_index("c")
    wid = sid * _NC + cid

    @pl.when(sid == 0)
    def _():
        pltpu.sync_copy(f_hbm.at[pl.ds(0, _TROWS)], f_sp)

    plsc.subcore_barrier()

    def chunk_body(c, carry):
        g = wid * _CPW + c
        pltpu.sync_copy(idxh.at[g], idx_v)

        def fire(j, cc):
            pltpu.async_copy(f_sp.at[idx_v.at[j]], gth_v.at[pl.ds(j * _CH, _CH)], sem)
            return cc

        lax.fori_loop(0, _K, fire, 0)
        pltpu.make_async_copy(f_hbm.at[pl.ds(0, _K * _CH)], gth_v, sem).wait()
        pltpu.sync_copy(gth_v, gth_hbm.at[pl.ds(g * _K * _CH, _K * _CH)])
        return carry

    lax.fori_loop(0, _CPW, chunk_body, 0)


@functools.cache
def _sc_gather_kernel():
    return functools.partial(
        pl.kernel,
        out_type=jax.ShapeDtypeStruct((_R, _C), jnp.float32),
        mesh=plsc.VectorSubcoreMesh(
            core_axis_name="c", subcore_axis_name="s",
            num_cores=_NC, num_subcores=_NS),
        scratch_types=[
            pltpu.VMEM_SHARED((_TROWS, _C), jnp.float32),
            pltpu.VMEM((_K, _CH), jnp.int32),
            pltpu.VMEM((_K * _CH, _C), jnp.float32),
            pltpu.SemaphoreType.DMA,
        ],
        compiler_params=pltpu.CompilerParams(use_tc_tiling_on_sc=False),
    )(_sc_body)


def kernel(features, neighbor_idx, W1, W2, gamma1, beta1, gamma2, beta2):
    nbr = neighbor_idx.astype(jnp.int32)
    fidx = jnp.where(nbr >= 0, nbr, _N)
    fidx = jnp.pad(fidx, ((0, _NPAD - _N), (0, 0)), constant_values=_N)
    idx3d = fidx.reshape(-1, _K, _CH)

    f_pad = jnp.pad(features, ((0, _NPAD - _N), (0, 0)))
    w1s = W1.reshape(_K * _C, _C)
    w2s = W2.reshape(_K * _C, _C)
    g1v = gamma1.reshape(1, _C)
    b1v = beta1.reshape(1, _C)
    g2v = gamma2.reshape(1, _C)
    b2v = beta2.reshape(1, _C)

    gth1 = _sc_gather_kernel()(f_pad, idx3d).reshape(_NPAD, _K * _C)
    out1, st1 = _mm_stats(gth1, w1s)
    h = _affine_relu(out1, st1, g1v, b1v)
    gth2 = _sc_gather_kernel()(h, idx3d).reshape(_NPAD, _K * _C)
    out2, st2 = _mm_stats(gth2, w2s)
    y = _final(out2, st2, g2v, b2v, f_pad)
    return y[:_N]
